# Initial kernel scaffold; baseline (speedup 1.0000x reference)
import jax
import jax.numpy as jnp
from jax.experimental import pallas as pl


def kernel(pred_deltas, objectness, anchors):
    raise NotImplementedError("write your pallas kernel here")


# speedup vs baseline: 357.7941x; 357.7941x over previous
import jax
import jax.numpy as jnp
from jax import lax
from jax.experimental import pallas as pl

_N = 20000
_MAX_OUT = 1000
_IMG = 1024.0
_IOU_T = 0.5


def _decode_body(d_ref, a_ref, o_ref):
    a0 = a_ref[0:1, :]
    a1 = a_ref[1:2, :]
    a2 = a_ref[2:3, :]
    a3 = a_ref[3:4, :]
    w = a2 - a0
    h = a3 - a1
    cx = a0 + 0.5 * w
    cy = a1 + 0.5 * h
    dx = d_ref[0:1, :]
    dy = d_ref[1:2, :]
    dw = d_ref[2:3, :]
    dh = d_ref[3:4, :]
    pcx = dx * w + cx
    pcy = dy * h + cy
    pw = jnp.exp(jnp.minimum(dw, 4.0)) * w
    ph = jnp.exp(jnp.minimum(dh, 4.0)) * h
    o_ref[0:1, :] = jnp.clip(pcx - 0.5 * pw, 0.0, _IMG)
    o_ref[1:2, :] = jnp.clip(pcy - 0.5 * ph, 0.0, _IMG)
    o_ref[2:3, :] = jnp.clip(pcx + 0.5 * pw, 0.0, _IMG)
    o_ref[3:4, :] = jnp.clip(pcy + 0.5 * ph, 0.0, _IMG)


def _nms_body(tbl_ref, ox1, oy1, ox2, oy2, osc):
    sub = lax.broadcasted_iota(jnp.int32, (8, 128), 0)
    lane = lax.broadcasted_iota(jnp.int32, (8, 128), 1)
    lin = sub * 128 + lane
    zf = jnp.zeros((8, 128), jnp.float32)

    def cond(st):
        c, kc, done = st[0], st[1], st[2]
        return (c < _N) & (kc < _MAX_OUT) & (done == 0)

    def body(st):
        c, kc, done, kx1, ky1, kx2, ky2, ks = st
        row = tbl_ref[pl.ds(c, 1), :]
        cx1 = row[0, 0]
        cy1 = row[0, 1]
        cx2 = row[0, 2]
        cy2 = row[0, 3]
        cs = row[0, 4]
        invalid = cs < 0.0
        ca = (cx2 - cx1) * (cy2 - cy1)
        xx1 = jnp.maximum(kx1, cx1)
        yy1 = jnp.maximum(ky1, cy1)
        xx2 = jnp.minimum(kx2, cx2)
        yy2 = jnp.minimum(ky2, cy2)
        inter = jnp.maximum(xx2 - xx1, 0.0) * jnp.maximum(yy2 - yy1, 0.0)
        karea = (kx2 - kx1) * (ky2 - ky1)
        iou = inter / ((karea + ca) - inter + 1e-9)
        sup = jnp.any((iou > _IOU_T) & (lin < kc))
        keep = jnp.logical_not(invalid) & jnp.logical_not(sup)
        at = (lin == kc) & keep
        kx1 = jnp.where(at, cx1, kx1)
        ky1 = jnp.where(at, cy1, ky1)
        kx2 = jnp.where(at, cx2, kx2)
        ky2 = jnp.where(at, cy2, ky2)
        ks = jnp.where(at, cs, ks)
        kc = kc + keep.astype(jnp.int32)
        done = invalid.astype(jnp.int32)
        return (c + 1, kc, done, kx1, ky1, kx2, ky2, ks)

    init = (jnp.int32(0), jnp.int32(0), jnp.int32(0), zf, zf, zf, zf, zf)
    st = lax.while_loop(cond, body, init)
    kc = st[1]
    valid = (lin < kc).astype(jnp.float32)
    ox1[:, :] = st[3] * valid
    oy1[:, :] = st[4] * valid
    ox2[:, :] = st[5] * valid
    oy2[:, :] = st[6] * valid
    osc[:, :] = st[7] * valid


def kernel(pred_deltas, objectness, anchors):
    d_t = pred_deltas.T
    a_t = anchors.T
    boxes4 = pl.pallas_call(
        _decode_body,
        out_shape=jax.ShapeDtypeStruct((4, _N), jnp.float32),
    )(d_t, a_t)

    ws = boxes4[2] - boxes4[0]
    hs = boxes4[3] - boxes4[1]
    valid = (ws >= 1.0) & (hs >= 1.0)
    scores = jnp.where(valid, jax.nn.sigmoid(objectness), -1.0)
    order = jnp.argsort(-scores)

    tbl = jnp.concatenate(
        [boxes4.T, scores[:, None], jnp.zeros((_N, 3), jnp.float32)], axis=1
    )
    tbl_s = jnp.take(tbl, order, axis=0)

    out8 = jax.ShapeDtypeStruct((8, 128), jnp.float32)
    x1, y1, x2, y2, sc = pl.pallas_call(
        _nms_body,
        out_shape=(out8, out8, out8, out8, out8),
    )(tbl_s)

    cols = [a.reshape(1024)[:_MAX_OUT] for a in (x1, y1, x2, y2, sc)]
    return jnp.stack(cols, axis=1)


# speedup vs baseline: 514.0346x; 1.4367x over previous
import jax
import jax.numpy as jnp
from jax import lax
from jax.experimental import pallas as pl

_N = 20000
_MAX_OUT = 1000
_IMG = 1024.0
_IOU_T = 0.5


def _decode_body(d_ref, a_ref, o_ref):
    a0 = a_ref[0:1, :]
    a1 = a_ref[1:2, :]
    a2 = a_ref[2:3, :]
    a3 = a_ref[3:4, :]
    w = a2 - a0
    h = a3 - a1
    cx = a0 + 0.5 * w
    cy = a1 + 0.5 * h
    dx = d_ref[0:1, :]
    dy = d_ref[1:2, :]
    dw = d_ref[2:3, :]
    dh = d_ref[3:4, :]
    pcx = dx * w + cx
    pcy = dy * h + cy
    pw = jnp.exp(jnp.minimum(dw, 4.0)) * w
    ph = jnp.exp(jnp.minimum(dh, 4.0)) * h
    o_ref[0:1, :] = jnp.clip(pcx - 0.5 * pw, 0.0, _IMG)
    o_ref[1:2, :] = jnp.clip(pcy - 0.5 * ph, 0.0, _IMG)
    o_ref[2:3, :] = jnp.clip(pcx + 0.5 * pw, 0.0, _IMG)
    o_ref[3:4, :] = jnp.clip(pcy + 0.5 * ph, 0.0, _IMG)


_BLK = 8


def _nms_body(tbl_ref, ox1, oy1, ox2, oy2, osc):
    lane = lax.broadcasted_iota(jnp.int32, (8, 128), 1)
    lane0 = lane == 0
    zf = jnp.zeros((8, 128), jnp.float32)
    lin = lax.broadcasted_iota(jnp.int32, (8, 128), 0) * 128 + lane
    pos0 = (lin == 0).astype(jnp.float32)

    def rollpos(p):
        a = jnp.concatenate([p[:, -1:], p[:, :-1]], axis=1)
        b = jnp.concatenate([a[-1:, :], a[:-1, :]], axis=0)
        return jnp.where(lane0, b, a)

    def cond(st):
        c, kc, done = st[0], st[1], st[2]
        return (c < _N) & (kc < _MAX_OUT) & (done == 0)

    def body(st):
        c, kc, done, pos, lt, kx1, ky1, kx2, ky2, ks, karea = st
        rows = tbl_ref[pl.ds(c, _BLK), :]
        nkeep = jnp.int32(0)
        bad = jnp.bool_(False)
        for k in range(_BLK):
            cx1 = rows[k, 0]
            cy1 = rows[k, 1]
            cx2 = rows[k, 2]
            cy2 = rows[k, 3]
            cs = rows[k, 4]
            invalid = cs < 0.0
            ca = (cx2 - cx1) * (cy2 - cy1)
            xx1 = jnp.maximum(kx1, cx1)
            yy1 = jnp.maximum(ky1, cy1)
            xx2 = jnp.minimum(kx2, cx2)
            yy2 = jnp.minimum(ky2, cy2)
            inter = jnp.maximum(xx2 - xx1, 0.0) * jnp.maximum(yy2 - yy1, 0.0)
            iou = inter / ((karea + ca) - inter + 1e-9)
            sup = jnp.any((iou > _IOU_T) & (lt > 0.0))
            keep = jnp.logical_not(invalid) & jnp.logical_not(sup)
            at = (pos > 0.0) & keep
            kx1 = jnp.where(at, cx1, kx1)
            ky1 = jnp.where(at, cy1, ky1)
            kx2 = jnp.where(at, cx2, kx2)
            ky2 = jnp.where(at, cy2, ky2)
            ks = jnp.where(at, cs, ks)
            karea = jnp.where(at, ca, karea)
            lt = jnp.where(at, 1.0, lt)
            pos = jnp.where(keep, rollpos(pos), pos)
            nkeep = nkeep + keep.astype(jnp.int32)
            bad = bad | invalid
        return (c + _BLK, kc + nkeep, bad.astype(jnp.int32), pos, lt,
                kx1, ky1, kx2, ky2, ks, karea)

    init = (jnp.int32(0), jnp.int32(0), jnp.int32(0), pos0, zf,
            zf, zf, zf, zf, zf, zf)
    st = lax.while_loop(cond, body, init)
    valid = st[4]
    ox1[:, :] = st[5] * valid
    oy1[:, :] = st[6] * valid
    ox2[:, :] = st[7] * valid
    oy2[:, :] = st[8] * valid
    osc[:, :] = st[9] * valid


def kernel(pred_deltas, objectness, anchors):
    d_t = pred_deltas.T
    a_t = anchors.T
    boxes4 = pl.pallas_call(
        _decode_body,
        out_shape=jax.ShapeDtypeStruct((4, _N), jnp.float32),
    )(d_t, a_t)

    ws = boxes4[2] - boxes4[0]
    hs = boxes4[3] - boxes4[1]
    valid = (ws >= 1.0) & (hs >= 1.0)
    scores = jnp.where(valid, jax.nn.sigmoid(objectness), -1.0)
    order = jnp.argsort(-scores)

    tbl = jnp.concatenate(
        [boxes4.T, scores[:, None], jnp.zeros((_N, 3), jnp.float32)], axis=1
    )
    tbl_s = jnp.take(tbl, order, axis=0)

    out8 = jax.ShapeDtypeStruct((8, 128), jnp.float32)
    x1, y1, x2, y2, sc = pl.pallas_call(
        _nms_body,
        out_shape=(out8, out8, out8, out8, out8),
    )(tbl_s)

    cols = [a.reshape(1024)[:_MAX_OUT] for a in (x1, y1, x2, y2, sc)]
    return jnp.stack(cols, axis=1)
